# baseline (speedup 1.0000x reference)
import functools

import jax
import jax.numpy as jnp
from jax import lax
from jax.experimental import pallas as pl
from jax.experimental.pallas import tpu as pltpu
from jax.experimental.pallas import tpu_sc as plsc

NC = 2
NS = 16
NW = NC * NS
CHUNK = 100
NBUF = 4

LANES = 16


def _mesh():
    return plsc.VectorSubcoreMesh(
        core_axis_name="c", subcore_axis_name="s",
        num_cores=NC, num_subcores=NS)


def _sc_relayout(embT, V, D):
    VCHUNK = 128
    nfull = V // VCHUNK
    vtail = V - nfull * VCHUNK
    npair = nfull // 2
    nodd = nfull - npair * 2
    base_p, extra_p = npair // NW, npair % NW
    obw = VCHUNK * D

    @functools.partial(
        pl.kernel,
        out_type=jax.ShapeDtypeStruct((V * D,), jnp.float32),
        mesh=_mesh(),
        scratch_types=[
            pltpu.VMEM((D, VCHUNK), jnp.float32),
            pltpu.VMEM((D, VCHUNK), jnp.float32),
            pltpu.VMEM((D, VCHUNK + 1), jnp.float32),
            pltpu.VMEM((obw,), jnp.float32),
            pltpu.VMEM((obw,), jnp.float32),
            pltpu.VMEM((D, vtail), jnp.float32),
            pltpu.VMEM((vtail * D,), jnp.float32),
        ] + [pltpu.SemaphoreType.DMA] * 4,
        compiler_params=pltpu.CompilerParams(use_tc_tiling_on_sc=True,
                                             needs_layout_passes=False),
    )
    def relayout_kernel(embT_hbm, out_hbm, ib0, ib1, skw, ob0, ob1, ibt, obt,
                        si0, si1, so0, so1):
        ib = (ib0, ib1)
        ob = (ob0, ob1)
        wid = lax.axis_index("s") * NC + lax.axis_index("c")
        start = 2 * (wid * base_p + jnp.minimum(wid, extra_p))
        cnt = 2 * (base_p + jnp.where(wid < extra_p, 1, 0))
        sin = (si0, si1)
        sout = (so0, so1)

        dlane = lax.iota(jnp.int32, LANES)

        def start_in(k, b):
            pltpu.async_copy(
                embT_hbm.at[:, pl.ds((start + k) * VCHUNK, VCHUNK)],
                ib[b], sin[b])

        def wait_in(b):
            pltpu.make_async_copy(
                embT_hbm.at[:, pl.ds(0, VCHUNK)], ib[b], sin[b]).wait()

        def wait_out(b):
            pltpu.make_async_copy(
                ob[b], out_hbm.at[pl.ds(0, obw)], sout[b]).wait()

        dvecs = [c * LANES + dlane for c in range(D // LANES)]

        def transpose(src, dst, width, skew):
            if skew:
                def copy_body(d, carry):
                    for vb in range(width // LANES):
                        skw[d, pl.ds(vb * LANES, LANES)] = (
                            src[d, pl.ds(vb * LANES, LANES)])
                    return carry
                lax.fori_loop(0, D, copy_body, 0, unroll=2)
                src = skw

            def body(v, vv):
                for c in range(D // LANES):
                    vals = plsc.load_gather(src, [dvecs[c], vv])
                    dst[pl.ds(v * D + c * LANES, LANES)] = vals
                return vv + 1
            lax.fori_loop(0, width, body, jnp.zeros((LANES,), jnp.int32),
                          unroll=4)

        start_in(0, 0)

        def loop_body(g, carry):
            for b in (0, 1):
                k = 2 * g + b

                @pl.when(k + 1 < cnt)
                def _():
                    start_in(k + 1, 1 - b)

                wait_in(b)

                @pl.when(k >= 2)
                def _():
                    wait_out(b)

                transpose(ib[b], ob[b], VCHUNK, skew=True)
                pltpu.async_copy(
                    ob[b], out_hbm.at[pl.ds((start + k) * obw, obw)],
                    sout[b])
            return carry

        lax.fori_loop(0, cnt // 2, loop_body, 0)
        wait_out(0)
        wait_out(1)

        if vtail:
            @pl.when(wid == NW - 1)
            def _():
                pltpu.sync_copy(
                    embT_hbm.at[:, pl.ds(nfull * VCHUNK, vtail)], ibt)
                transpose(ibt, obt, vtail, skew=False)
                pltpu.sync_copy(
                    obt, out_hbm.at[pl.ds(nfull * VCHUNK * D, vtail * D)])

    return relayout_kernel(embT)


def _sc_gather_maxpool(x2, emb, B, S, D):
    bpw = B // NW
    cpr = S // CHUNK
    cpw = bpw * cpr
    nlc = D // 16

    @functools.partial(
        pl.kernel,
        out_type=jax.ShapeDtypeStruct((B, D), jnp.float32),
        mesh=_mesh(),
        scratch_types=[
            pltpu.VMEM((cpw, CHUNK), jnp.int32),
            pltpu.VMEM((NBUF, S, D), jnp.float32),
            pltpu.VMEM((bpw, D), jnp.float32),
        ] + [pltpu.SemaphoreType.DMA] * NBUF,
        compiler_params=pltpu.CompilerParams(use_tc_tiling_on_sc=False),
    )
    def pool_kernel(x_hbm, emb_hbm, out_hbm, idx_v, rows_v, out_v, *sems):
        wid = lax.axis_index("s") * NC + lax.axis_index("c")
        pltpu.sync_copy(x_hbm.at[pl.ds(wid * cpw, cpw)], idx_v)

        def start_row(row, buf):
            for h in range(cpr):
                pltpu.async_copy(
                    emb_hbm.at[idx_v.at[row * cpr + h]],
                    rows_v.at[buf, pl.ds(h * CHUNK, CHUNK)],
                    sems[buf])

        def wait_row(buf):
            for h in range(cpr):
                pltpu.make_async_copy(
                    emb_hbm.at[idx_v.at[0]],
                    rows_v.at[buf, pl.ds(h * CHUNK, CHUNK)],
                    sems[buf]).wait()

        def reduce_row(row, buf):
            def body(j, accs):
                return tuple(
                    jnp.maximum(accs[c], rows_v[buf, j, pl.ds(c * 16, 16)])
                    for c in range(nlc))
            inits = tuple(rows_v[buf, 0, pl.ds(c * 16, 16)] for c in range(nlc))
            accs = lax.fori_loop(1, S, body, inits, unroll=4)
            for c in range(nlc):
                out_v[row, pl.ds(c * 16, 16)] = accs[c]

        for b in range(NBUF):
            start_row(b, b)

        def loop_body(g, carry):
            for b in range(NBUF):
                row = g * NBUF + b
                wait_row(b)
                reduce_row(row, b)

                @pl.when(row + NBUF < bpw)
                def _():
                    start_row(row + NBUF, b)
            return carry

        lax.fori_loop(0, bpw // NBUF, loop_body, 0)
        pltpu.sync_copy(out_v, out_hbm.at[pl.ds(wid * bpw, bpw)])

    return pool_kernel(x2, emb)


def _tc_linear(pooled, W, b2):
    def mm_body(p_ref, w_ref, b_ref, o_ref):
        o_ref[...] = lax.dot_general(
            p_ref[...], w_ref[...],
            (((1,), (1,)), ((), ())),
            preferred_element_type=jnp.float32) + b_ref[...]

    return pl.pallas_call(
        mm_body,
        out_shape=jax.ShapeDtypeStruct((pooled.shape[0], W.shape[0]),
                                       jnp.float32),
    )(pooled, W, b2)


@jax.jit
def kernel(x, emb, W, b):
    B, S = x.shape
    V, D = emb.shape
    x2 = x.astype(jnp.int32).reshape(B * S // CHUNK, CHUNK)
    emb_rows = _sc_relayout(jnp.swapaxes(emb, 0, 1), V, D).reshape(V, D)
    pooled = _sc_gather_maxpool(x2, emb_rows, B, S, D)
    return _tc_linear(pooled, W, b.reshape(1, -1))


# speedup vs baseline: 3.3035x; 3.3035x over previous
import functools

import jax
import jax.numpy as jnp
from jax import lax
from jax.experimental import pallas as pl
from jax.experimental.pallas import tpu as pltpu
from jax.experimental.pallas import tpu_sc as plsc

NC = 2
NS = 16
NW = NC * NS
CHUNK = 100
NBUF = 4

LANES = 16


def _mesh():
    return plsc.VectorSubcoreMesh(
        core_axis_name="c", subcore_axis_name="s",
        num_cores=NC, num_subcores=NS)


def _sc_relayout(embT, V, D):
    VCHUNK = 256
    nfull = V // VCHUNK
    vtail = V - nfull * VCHUNK
    npair = nfull // 2
    nodd = nfull - npair * 2
    base_p, extra_p = npair // NW, npair % NW
    obw = VCHUNK * D

    @functools.partial(
        pl.kernel,
        out_type=jax.ShapeDtypeStruct((V * D,), jnp.float32),
        mesh=_mesh(),
        scratch_types=[
            pltpu.VMEM((D, VCHUNK), jnp.float32),
            pltpu.VMEM((D, VCHUNK), jnp.float32),
            pltpu.VMEM((obw,), jnp.float32),
            pltpu.VMEM((obw,), jnp.float32),
            pltpu.VMEM((D, vtail), jnp.float32),
            pltpu.VMEM((vtail * D,), jnp.float32),
        ] + [pltpu.SemaphoreType.DMA] * 4,
        compiler_params=pltpu.CompilerParams(use_tc_tiling_on_sc=True,
                                             needs_layout_passes=False),
    )
    def relayout_kernel(embT_hbm, out_hbm, ib0, ib1, ob0, ob1, ibt, obt,
                        si0, si1, so0, so1):
        ib = (ib0, ib1)
        ob = (ob0, ob1)
        wid = lax.axis_index("s") * NC + lax.axis_index("c")
        start = 2 * (wid * base_p + jnp.minimum(wid, extra_p))
        cnt = 2 * (base_p + jnp.where(wid < extra_p, 1, 0))
        sin = (si0, si1)
        sout = (so0, so1)

        dlane = lax.iota(jnp.int32, LANES)

        def start_in(k, b):
            pltpu.async_copy(
                embT_hbm.at[:, pl.ds((start + k) * VCHUNK, VCHUNK)],
                ib[b], sin[b])

        def wait_in(b):
            pltpu.make_async_copy(
                embT_hbm.at[:, pl.ds(0, VCHUNK)], ib[b], sin[b]).wait()

        def wait_out(b):
            pltpu.make_async_copy(
                ob[b], out_hbm.at[pl.ds(0, obw)], sout[b]).wait()

        perms = [jnp.mod(dlane + s, LANES) for s in range(LANES)]

        def transpose(src, dst, width):
            for c in range(D // LANES):
                dvec = c * LANES + dlane

                def body(vb, carry):
                    v0 = vb * LANES
                    for s in range(LANES):
                        vv = v0 + perms[s]
                        vals = plsc.load_gather(src, [dvec, vv])
                        plsc.store_scatter(dst, [vv * D + dvec], vals)
                    return carry
                lax.fori_loop(0, width // LANES, body, 0)

        start_in(0, 0)

        def loop_body(g, carry):
            for b in (0, 1):
                k = 2 * g + b

                @pl.when(k + 1 < cnt)
                def _():
                    start_in(k + 1, 1 - b)

                wait_in(b)

                @pl.when(k >= 2)
                def _():
                    wait_out(b)

                transpose(ib[b], ob[b], VCHUNK)
                pltpu.async_copy(
                    ob[b], out_hbm.at[pl.ds((start + k) * obw, obw)],
                    sout[b])
            return carry

        lax.fori_loop(0, cnt // 2, loop_body, 0)
        wait_out(0)
        wait_out(1)

        if vtail:
            @pl.when(wid == NW - 1)
            def _():
                pltpu.sync_copy(
                    embT_hbm.at[:, pl.ds(nfull * VCHUNK, vtail)], ibt)
                transpose(ibt, obt, vtail)
                pltpu.sync_copy(
                    obt, out_hbm.at[pl.ds(nfull * VCHUNK * D, vtail * D)])

    return relayout_kernel(embT)


def _sc_gather_maxpool(x2, emb, B, S, D):
    bpw = B // NW
    cpr = S // CHUNK
    cpw = bpw * cpr
    nlc = D // 16

    @functools.partial(
        pl.kernel,
        out_type=jax.ShapeDtypeStruct((B, D), jnp.float32),
        mesh=_mesh(),
        scratch_types=[
            pltpu.VMEM((cpw, CHUNK), jnp.int32),
            pltpu.VMEM((NBUF, S, D), jnp.float32),
            pltpu.VMEM((bpw, D), jnp.float32),
        ] + [pltpu.SemaphoreType.DMA] * NBUF,
        compiler_params=pltpu.CompilerParams(use_tc_tiling_on_sc=False),
    )
    def pool_kernel(x_hbm, emb_hbm, out_hbm, idx_v, rows_v, out_v, *sems):
        wid = lax.axis_index("s") * NC + lax.axis_index("c")
        pltpu.sync_copy(x_hbm.at[pl.ds(wid * cpw, cpw)], idx_v)

        def start_row(row, buf):
            for h in range(cpr):
                pltpu.async_copy(
                    emb_hbm.at[idx_v.at[row * cpr + h]],
                    rows_v.at[buf, pl.ds(h * CHUNK, CHUNK)],
                    sems[buf])

        def wait_row(buf):
            for h in range(cpr):
                pltpu.make_async_copy(
                    emb_hbm.at[idx_v.at[0]],
                    rows_v.at[buf, pl.ds(h * CHUNK, CHUNK)],
                    sems[buf]).wait()

        def reduce_row(row, buf):
            def body(j, accs):
                return tuple(
                    jnp.maximum(accs[c], rows_v[buf, j, pl.ds(c * 16, 16)])
                    for c in range(nlc))
            inits = tuple(rows_v[buf, 0, pl.ds(c * 16, 16)] for c in range(nlc))
            accs = lax.fori_loop(1, S, body, inits, unroll=4)
            for c in range(nlc):
                out_v[row, pl.ds(c * 16, 16)] = accs[c]

        for b in range(NBUF):
            start_row(b, b)

        def loop_body(g, carry):
            for b in range(NBUF):
                row = g * NBUF + b
                wait_row(b)
                reduce_row(row, b)

                @pl.when(row + NBUF < bpw)
                def _():
                    start_row(row + NBUF, b)
            return carry

        lax.fori_loop(0, bpw // NBUF, loop_body, 0)
        pltpu.sync_copy(out_v, out_hbm.at[pl.ds(wid * bpw, bpw)])

    return pool_kernel(x2, emb)


def _tc_linear(pooled, W, b2):
    def mm_body(p_ref, w_ref, b_ref, o_ref):
        o_ref[...] = lax.dot_general(
            p_ref[...], w_ref[...],
            (((1,), (1,)), ((), ())),
            preferred_element_type=jnp.float32) + b_ref[...]

    return pl.pallas_call(
        mm_body,
        out_shape=jax.ShapeDtypeStruct((pooled.shape[0], W.shape[0]),
                                       jnp.float32),
    )(pooled, W, b2)


@jax.jit
def kernel(x, emb, W, b):
    B, S = x.shape
    V, D = emb.shape
    x2 = x.astype(jnp.int32).reshape(B * S // CHUNK, CHUNK)
    emb_rows = _sc_relayout(jnp.swapaxes(emb, 0, 1), V, D).reshape(V, D)
    pooled = _sc_gather_maxpool(x2, emb_rows, B, S, D)
    return _tc_linear(pooled, W, b.reshape(1, -1))


# speedup vs baseline: 3.4041x; 1.0305x over previous
import functools

import jax
import jax.numpy as jnp
from jax import lax
from jax.experimental import pallas as pl
from jax.experimental.pallas import tpu as pltpu
from jax.experimental.pallas import tpu_sc as plsc

NC = 2
NS = 16
NW = NC * NS
CHUNK = 100
NBUF = 4

LANES = 16


def _mesh():
    return plsc.VectorSubcoreMesh(
        core_axis_name="c", subcore_axis_name="s",
        num_cores=NC, num_subcores=NS)


def _sc_relayout(embT, V, D):
    VCHUNK = 128
    nfull = V // VCHUNK
    vtail = V - nfull * VCHUNK
    npair = nfull // 2
    nodd = nfull - npair * 2
    base_p, extra_p = npair // NW, npair % NW
    obw = VCHUNK * D

    @functools.partial(
        pl.kernel,
        out_type=jax.ShapeDtypeStruct((V * D,), jnp.float32),
        mesh=_mesh(),
        scratch_types=[
            pltpu.VMEM((D, VCHUNK), jnp.float32),
            pltpu.VMEM((D, VCHUNK), jnp.float32),
            pltpu.VMEM((obw,), jnp.float32),
            pltpu.VMEM((obw,), jnp.float32),
            pltpu.VMEM((D, vtail), jnp.float32),
            pltpu.VMEM((vtail * D,), jnp.float32),
        ] + [pltpu.SemaphoreType.DMA] * 4,
        compiler_params=pltpu.CompilerParams(use_tc_tiling_on_sc=True,
                                             needs_layout_passes=False),
    )
    def relayout_kernel(embT_hbm, out_hbm, ib0, ib1, ob0, ob1, ibt, obt,
                        si0, si1, so0, so1):
        ib = (ib0, ib1)
        ob = (ob0, ob1)
        wid = lax.axis_index("s") * NC + lax.axis_index("c")
        start = 2 * (wid * base_p + jnp.minimum(wid, extra_p))
        cnt = 2 * (base_p + jnp.where(wid < extra_p, 1, 0))
        sin = (si0, si1)
        sout = (so0, so1)

        dlane = lax.iota(jnp.int32, LANES)

        def start_in(k, b):
            pltpu.async_copy(
                embT_hbm.at[:, pl.ds((start + k) * VCHUNK, VCHUNK)],
                ib[b], sin[b])

        def wait_in(b):
            pltpu.make_async_copy(
                embT_hbm.at[:, pl.ds(0, VCHUNK)], ib[b], sin[b]).wait()

        def wait_out(b):
            pltpu.make_async_copy(
                ob[b], out_hbm.at[pl.ds(0, obw)], sout[b]).wait()

        perms = [jnp.mod(dlane + s, LANES) for s in range(LANES)]

        def transpose(src, dst, width):
            for c in range(D // LANES):
                dvec = c * LANES + dlane

                def body(vb, carry):
                    v0 = vb * LANES
                    for s in range(LANES):
                        vv = v0 + perms[s]
                        vals = plsc.load_gather(src, [dvec, vv])
                        plsc.store_scatter(dst, [vv * D + dvec], vals)
                    return carry
                lax.fori_loop(0, width // LANES, body, 0)

        start_in(0, 0)

        def loop_body(g, carry):
            for b in (0, 1):
                k = 2 * g + b

                @pl.when(k + 1 < cnt)
                def _():
                    start_in(k + 1, 1 - b)

                wait_in(b)

                @pl.when(k >= 2)
                def _():
                    wait_out(b)

                transpose(ib[b], ob[b], VCHUNK)
                pltpu.async_copy(
                    ob[b], out_hbm.at[pl.ds((start + k) * obw, obw)],
                    sout[b])
            return carry

        lax.fori_loop(0, cnt // 2, loop_body, 0)
        wait_out(0)
        wait_out(1)

        if vtail:
            @pl.when(wid == NW - 1)
            def _():
                pltpu.sync_copy(
                    embT_hbm.at[:, pl.ds(nfull * VCHUNK, vtail)], ibt)
                transpose(ibt, obt, vtail)
                pltpu.sync_copy(
                    obt, out_hbm.at[pl.ds(nfull * VCHUNK * D, vtail * D)])

    return relayout_kernel(embT)


def _sc_gather_maxpool(x2, emb, B, S, D):
    bpw = B // NW
    cpr = S // CHUNK
    cpw = bpw * cpr
    nlc = D // 16

    @functools.partial(
        pl.kernel,
        out_type=jax.ShapeDtypeStruct((B, D), jnp.float32),
        mesh=_mesh(),
        scratch_types=[
            pltpu.VMEM((cpw, CHUNK), jnp.int32),
            pltpu.VMEM((NBUF, S, D), jnp.float32),
            pltpu.VMEM((bpw, D), jnp.float32),
        ] + [pltpu.SemaphoreType.DMA] * NBUF,
        compiler_params=pltpu.CompilerParams(use_tc_tiling_on_sc=False),
    )
    def pool_kernel(x_hbm, emb_hbm, out_hbm, idx_v, rows_v, out_v, *sems):
        wid = lax.axis_index("s") * NC + lax.axis_index("c")
        pltpu.sync_copy(x_hbm.at[pl.ds(wid * cpw, cpw)], idx_v)

        def start_row(row, buf):
            for h in range(cpr):
                pltpu.async_copy(
                    emb_hbm.at[idx_v.at[row * cpr + h]],
                    rows_v.at[buf, pl.ds(h * CHUNK, CHUNK)],
                    sems[buf])

        def wait_row(buf):
            for h in range(cpr):
                pltpu.make_async_copy(
                    emb_hbm.at[idx_v.at[0]],
                    rows_v.at[buf, pl.ds(h * CHUNK, CHUNK)],
                    sems[buf]).wait()

        def reduce_row(row, buf):
            def body(j, accs):
                return tuple(
                    jnp.maximum(accs[c], rows_v[buf, j, pl.ds(c * 16, 16)])
                    for c in range(nlc))
            inits = tuple(rows_v[buf, 0, pl.ds(c * 16, 16)] for c in range(nlc))
            accs = lax.fori_loop(1, S, body, inits, unroll=4)
            for c in range(nlc):
                out_v[row, pl.ds(c * 16, 16)] = accs[c]

        for b in range(NBUF):
            start_row(b, b)

        def loop_body(g, carry):
            for b in range(NBUF):
                row = g * NBUF + b
                wait_row(b)
                reduce_row(row, b)

                @pl.when(row + NBUF < bpw)
                def _():
                    start_row(row + NBUF, b)
            return carry

        lax.fori_loop(0, bpw // NBUF, loop_body, 0)
        pltpu.sync_copy(out_v, out_hbm.at[pl.ds(wid * bpw, bpw)])

    return pool_kernel(x2, emb)


def _tc_linear(pooled, W, b2):
    def mm_body(p_ref, w_ref, b_ref, o_ref):
        o_ref[...] = lax.dot_general(
            p_ref[...], w_ref[...],
            (((1,), (1,)), ((), ())),
            preferred_element_type=jnp.float32) + b_ref[...]

    return pl.pallas_call(
        mm_body,
        out_shape=jax.ShapeDtypeStruct((pooled.shape[0], W.shape[0]),
                                       jnp.float32),
    )(pooled, W, b2)


@jax.jit
def kernel(x, emb, W, b):
    B, S = x.shape
    V, D = emb.shape
    x2 = x.astype(jnp.int32).reshape(B * S // CHUNK, CHUNK)
    emb_rows = _sc_relayout(jnp.swapaxes(emb, 0, 1), V, D).reshape(V, D)
    pooled = _sc_gather_maxpool(x2, emb_rows, B, S, D)
    return _tc_linear(pooled, W, b.reshape(1, -1))
